# baseline (speedup 1.0000x reference)
import functools

import jax
import jax.numpy as jnp
from jax import lax
from jax.experimental import pallas as pl
from jax.experimental.pallas import tpu as pltpu
from jax.experimental.pallas import tpu_sc as plsc

BATCH = 4096
SEQ = 50
TOKEN_DIM = 64
NUM_CORES = 2
NUM_SUBCORES = 16
NUM_WORKERS = NUM_CORES * NUM_SUBCORES
B_PER_W = BATCH // NUM_WORKERS
ROWS_PER_WORKER = B_PER_W * SEQ
NUM_CHUNKS = 16
CHUNK_B = B_PER_W // NUM_CHUNKS
CHUNK = CHUNK_B * SEQ
NBUF = 4
AHEAD = 2

_mesh = plsc.VectorSubcoreMesh(core_axis_name="c", subcore_axis_name="s")


@functools.partial(
    pl.kernel,
    out_type=jax.ShapeDtypeStruct((BATCH, SEQ, TOKEN_DIM), jnp.float32),
    mesh=_mesh,
    compiler_params=pltpu.CompilerParams(use_tc_tiling_on_sc=False),
    scratch_types=[
        pltpu.VMEM((ROWS_PER_WORKER,), jnp.int32),
        *[pltpu.VMEM((CHUNK, TOKEN_DIM), jnp.float32) for _ in range(NBUF)],
        *[pltpu.SemaphoreType.DMA for _ in range(2 * NBUF)],
    ],
)
def _sc_gather(table_hbm, idx_hbm, out_hbm, idx_v, *rest):
    bufs = rest[:NBUF]
    gsems = rest[NBUF : 2 * NBUF]
    osems = rest[2 * NBUF :]

    wid = lax.axis_index("s") * NUM_CORES + lax.axis_index("c")
    b0 = wid * B_PER_W
    pltpu.sync_copy(idx_hbm.at[pl.ds(wid * ROWS_PER_WORKER, ROWS_PER_WORKER)], idx_v)
    idx_flat = idx_v

    def start_gather(c):
        b = c % NBUF
        return pltpu.async_copy(
            table_hbm.at[idx_flat.at[pl.ds(c * CHUNK, CHUNK)]], bufs[b], gsems[b]
        )

    def start_out(c):
        b = c % NBUF
        copies = []
        for k in range(CHUNK_B):
            copies.append(
                pltpu.async_copy(
                    bufs[b].at[pl.ds(k * SEQ, SEQ)],
                    out_hbm.at[b0 + c * CHUNK_B + k],
                    osems[b],
                )
            )
        return copies

    g = [None] * NBUF
    o = [None] * NBUF
    for c in range(AHEAD):
        g[c] = start_gather(c)
    for c in range(NUM_CHUNKS):
        nc = c + AHEAD
        if nc < NUM_CHUNKS:
            b2 = nc % NBUF
            if o[b2] is not None:
                for cp in o[b2]:
                    cp.wait()
            g[b2] = start_gather(nc)
        b = c % NBUF
        g[b].wait()
        o[b] = start_out(c)
    for b in range(NBUF):
        if o[b] is not None:
            for cp in o[b]:
                cp.wait()


def kernel(indices, embedding_weight):
    flat = indices.reshape(-1).astype(jnp.int32)
    return _sc_gather(embedding_weight, flat)
